# baseline (speedup 1.0000x reference)
import functools

import jax
import jax.numpy as jnp
from jax import lax
from jax.experimental import pallas as pl
from jax.experimental.pallas import tpu as pltpu
from jax.experimental.pallas import tpu_sc as plsc

_L = 16


@functools.lru_cache(maxsize=None)
def _build(B, D):
    info = plsc.get_sparse_core_info()
    NC, NS = info.num_cores, info.num_subcores
    NW = NC * NS
    assert B % NW == 0 and D % _L == 0
    BPW = B // NW
    CH = 128
    NCH = BPW // CH
    NDC = D // _L
    assert BPW % CH == 0 and BPW % _L == 0

    mesh = plsc.VectorSubcoreMesh(core_axis_name="c", subcore_axis_name="s")
    cparams = pltpu.CompilerParams(
        needs_layout_passes=False, use_tc_tiling_on_sc=False
    )
    @functools.partial(
        pl.kernel,
        out_type=jax.ShapeDtypeStruct((B,), jnp.float32),
        mesh=mesh,
        compiler_params=cparams,
        scratch_types=[
            pltpu.VMEM((BPW,), jnp.int32),
            pltpu.VMEM((BPW,), jnp.int32),
            pltpu.VMEM((BPW, D), jnp.float32),
            pltpu.VMEM((BPW, D), jnp.float32),
            pltpu.VMEM((BPW,), jnp.float32),
            pltpu.VMEM((BPW,), jnp.float32),
            pltpu.VMEM((BPW,), jnp.float32),
            pltpu.VMEM((_L,), jnp.float32),
            pltpu.SemaphoreType.DMA,
        ],
    )
    def mf(uids_hbm, iids_hbm, tabs_hbm, biases_hbm, gb_hbm,
           out_hbm, uid_v, iid_v, urows, irows, ubr, ibr, out_v, gb_v, sem):
        wid = lax.axis_index("s") * NC + lax.axis_index("c")
        base = wid * BPW
        pltpu.sync_copy(gb_hbm, gb_v)
        pltpu.sync_copy(uids_hbm.at[pl.ds(base, BPW)], uid_v)
        pltpu.sync_copy(iids_hbm.at[pl.ds(base, BPW)], iid_v)
        copies = []
        for k in range(NCH):
            sl = pl.ds(k * CH, CH)
            copies.append(pltpu.async_copy(tabs_hbm.at[uid_v.at[sl]], urows.at[sl], sem))
            copies.append(pltpu.async_copy(tabs_hbm.at[iid_v.at[sl]], irows.at[sl], sem))
            copies.append(pltpu.async_copy(biases_hbm.at[uid_v.at[sl]], ubr.at[sl], sem))
            copies.append(pltpu.async_copy(biases_hbm.at[iid_v.at[sl]], ibr.at[sl], sem))
        for c in copies:
            c.wait()

        gbv = gb_v[pl.ds(0, _L)]
        lane = lax.iota(jnp.int32, _L)

        def group(g, carry):
            rbase = g * _L
            acc = ubr[pl.ds(rbase, _L)] + ibr[pl.ds(rbase, _L)] + gbv
            for j in range(_L):
                r = rbase + j
                s = urows[r, pl.ds(0, _L)] * irows[r, pl.ds(0, _L)]
                for cidx in range(1, NDC):
                    co = cidx * _L
                    s = s + urows[r, pl.ds(co, _L)] * irows[r, pl.ds(co, _L)]
                acc = acc + jnp.where(lane == j, jnp.sum(s), 0.0)
            out_v[pl.ds(rbase, _L)] = acc
            return carry

        lax.fori_loop(0, BPW // _L, group, 0)
        pltpu.sync_copy(out_v, out_hbm.at[pl.ds(base, BPW)])

    return mf


def kernel(user_ids, item_ids, user_table, item_table, user_bias, item_bias,
           global_bias):
    B = user_ids.shape[0]
    mf = _build(B, user_table.shape[1])
    tabs = jnp.concatenate([user_table, item_table], axis=0)
    biases = jnp.concatenate([user_bias.reshape(-1), item_bias.reshape(-1)])
    iids_off = item_ids.astype(jnp.int32) + jnp.int32(user_table.shape[0])
    return mf(user_ids.astype(jnp.int32), iids_off, tabs, biases,
              jnp.broadcast_to(global_bias, (_L,)))


# speedup vs baseline: 1.4714x; 1.4714x over previous
import functools

import jax
import jax.numpy as jnp
from jax import lax
from jax.experimental import pallas as pl
from jax.experimental.pallas import tpu as pltpu
from jax.experimental.pallas import tpu_sc as plsc

_L = 16


@functools.lru_cache(maxsize=None)
def _build(B, D):
    info = plsc.get_sparse_core_info()
    NC, NS = info.num_cores, info.num_subcores
    NW = NC * NS
    assert B % NW == 0 and D % _L == 0
    BPW = B // NW
    CH = 128
    NCH = BPW // CH
    NDC = D // _L
    GPC = CH // _L
    assert BPW % CH == 0 and BPW % _L == 0

    mesh = plsc.VectorSubcoreMesh(core_axis_name="c", subcore_axis_name="s")
    cparams = pltpu.CompilerParams(
        needs_layout_passes=False, use_tc_tiling_on_sc=True
    )

    @functools.partial(
        pl.kernel,
        out_type=jax.ShapeDtypeStruct((B,), jnp.float32),
        mesh=mesh,
        compiler_params=cparams,
        scratch_types=[
            pltpu.VMEM((BPW,), jnp.int32),
            pltpu.VMEM((BPW,), jnp.int32),
            pltpu.VMEM((BPW,), jnp.int32),
            pltpu.VMEM((BPW,), jnp.int32),
            pltpu.VMEM((2, CH, 2 * D), jnp.float32),
            pltpu.VMEM((2, CH, 2 * D), jnp.float32),
            pltpu.VMEM((BPW,), jnp.float32),
            pltpu.VMEM((BPW,), jnp.float32),
            pltpu.VMEM((BPW,), jnp.float32),
            pltpu.VMEM((_L,), jnp.float32),
            pltpu.SemaphoreType.DMA,
            pltpu.SemaphoreType.DMA,
            pltpu.SemaphoreType.DMA,
        ],
    )
    def mf(uids_hbm, iids_hbm, utab_hbm, itab_hbm, ub_hbm, ib_hbm, gb_hbm,
           out_hbm, uid_v, iid_v, uhalf, ihalf, ubuf, ibuf, ubr, ibr,
           out_v, gb_v, sem0, sem1, semb):
        wid = lax.axis_index("s") * NC + lax.axis_index("c")
        base = wid * BPW
        pltpu.sync_copy(gb_hbm, gb_v)
        pltpu.sync_copy(uids_hbm.at[pl.ds(base, BPW)], uid_v)
        pltpu.sync_copy(iids_hbm.at[pl.ds(base, BPW)], iid_v)
        for c in range(BPW // _L):
            sl = pl.ds(c * _L, _L)
            uhalf[sl] = lax.shift_right_logical(uid_v[sl], 1)
            ihalf[sl] = lax.shift_right_logical(iid_v[sl], 1)

        bias_copies = []
        for k in range(NCH):
            sl = pl.ds(k * CH, CH)
            bias_copies.append(
                pltpu.async_copy(ub_hbm.at[uid_v.at[sl]], ubr.at[sl], semb))
            bias_copies.append(
                pltpu.async_copy(ib_hbm.at[iid_v.at[sl]], ibr.at[sl], semb))

        sems = [sem0, sem1]

        def issue(k):
            sl = pl.ds(k * CH, CH)
            bsl = k % 2
            return (
                pltpu.async_copy(utab_hbm.at[uhalf.at[sl]], ubuf.at[bsl], sems[bsl]),
                pltpu.async_copy(itab_hbm.at[ihalf.at[sl]], ibuf.at[bsl], sems[bsl]),
            )

        inflight = issue(0)
        for c in bias_copies:
            c.wait()
        gbv = gb_v[pl.ds(0, _L)]
        lane = lax.iota(jnp.int32, _L)

        for k in range(NCH):
            cu, ci = inflight
            if k + 1 < NCH:
                nxt = issue(k + 1)
            cu.wait()
            ci.wait()
            if k + 1 < NCH:
                inflight = nxt
            ub2 = ubuf.at[k % 2]
            ib2 = ibuf.at[k % 2]

            def group(g, carry, _k=k, _ub2=ub2, _ib2=ib2):
                lbase = g * _L
                rbase = _k * CH + lbase
                ucolv = (uid_v[pl.ds(rbase, _L)] & 1) * D
                icolv = (iid_v[pl.ds(rbase, _L)] & 1) * D
                acc = ubr[pl.ds(rbase, _L)] + ibr[pl.ds(rbase, _L)] + gbv
                for j in range(_L):
                    lr = lbase + j
                    uc = ucolv[j]
                    ic = icolv[j]
                    s = _ub2[lr, pl.ds(uc, _L)] * _ib2[lr, pl.ds(ic, _L)]
                    for cidx in range(1, NDC):
                        co = cidx * _L
                        s = s + (_ub2[lr, pl.ds(uc + co, _L)]
                                 * _ib2[lr, pl.ds(ic + co, _L)])
                    acc = acc + jnp.where(lane == j, jnp.sum(s), 0.0)
                out_v[pl.ds(rbase, _L)] = acc
                return carry

            lax.fori_loop(0, GPC, group, 0)

        pltpu.sync_copy(out_v, out_hbm.at[pl.ds(base, BPW)])

    return mf


def kernel(user_ids, item_ids, user_table, item_table, user_bias, item_bias,
           global_bias):
    B = user_ids.shape[0]
    VU, D = user_table.shape
    VI = item_table.shape[0]
    assert VU % 2 == 0 and VI % 2 == 0
    mf = _build(B, D)
    return mf(user_ids.astype(jnp.int32), item_ids.astype(jnp.int32),
              user_table.reshape(VU // 2, 2 * D),
              item_table.reshape(VI // 2, 2 * D),
              user_bias.reshape(-1), item_bias.reshape(-1),
              jnp.broadcast_to(global_bias, (_L,)))


# speedup vs baseline: 1.5240x; 1.0358x over previous
import functools

import jax
import jax.numpy as jnp
from jax import lax
from jax.experimental import pallas as pl
from jax.experimental.pallas import tpu as pltpu
from jax.experimental.pallas import tpu_sc as plsc

_L = 16


@functools.lru_cache(maxsize=None)
def _build(B, D):
    info = plsc.get_sparse_core_info()
    NC, NS = info.num_cores, info.num_subcores
    NW = NC * NS
    assert B % NW == 0 and D % _L == 0
    BPW = B // NW
    CH = 128
    NCH = BPW // CH
    NDC = D // _L
    assert BPW % CH == 0 and BPW % _L == 0

    mesh = plsc.VectorSubcoreMesh(core_axis_name="c", subcore_axis_name="s")
    cparams = pltpu.CompilerParams(
        needs_layout_passes=False, use_tc_tiling_on_sc=False
    )

    @functools.partial(
        pl.kernel,
        out_type=jax.ShapeDtypeStruct((B,), jnp.float32),
        mesh=mesh,
        compiler_params=cparams,
        scratch_types=[
            pltpu.VMEM((BPW,), jnp.int32),
            pltpu.VMEM((BPW,), jnp.int32),
            pltpu.VMEM((BPW, D), jnp.float32),
            pltpu.VMEM((BPW, D), jnp.float32),
            pltpu.VMEM((BPW,), jnp.float32),
            pltpu.VMEM((BPW,), jnp.float32),
            pltpu.VMEM((BPW,), jnp.float32),
            pltpu.VMEM((_L,), jnp.float32),
            pltpu.SemaphoreType.DMA,
        ],
    )
    def mf(uids_hbm, iids_hbm, utab_hbm, itab_hbm, ub_hbm, ib_hbm, gb_hbm,
           out_hbm, uid_v, iid_v, urows, irows, ubr, ibr, out_v, gb_v, sem):
        wid = lax.axis_index("s") * NC + lax.axis_index("c")
        base = wid * BPW
        pltpu.sync_copy(gb_hbm, gb_v)
        pltpu.sync_copy(uids_hbm.at[pl.ds(base, BPW)], uid_v)
        pltpu.sync_copy(iids_hbm.at[pl.ds(base, BPW)], iid_v)
        copies = []
        for k in range(NCH):
            sl = pl.ds(k * CH, CH)
            copies.append(pltpu.async_copy(utab_hbm.at[uid_v.at[sl]], urows.at[sl], sem))
            copies.append(pltpu.async_copy(itab_hbm.at[iid_v.at[sl]], irows.at[sl], sem))
            copies.append(pltpu.async_copy(ub_hbm.at[uid_v.at[sl]], ubr.at[sl], sem))
            copies.append(pltpu.async_copy(ib_hbm.at[iid_v.at[sl]], ibr.at[sl], sem))
        for c in copies:
            c.wait()

        gbv = gb_v[pl.ds(0, _L)]
        lane = lax.iota(jnp.int32, _L)

        def group(g, carry):
            rbase = g * _L
            acc = ubr[pl.ds(rbase, _L)] + ibr[pl.ds(rbase, _L)] + gbv
            for j in range(_L):
                r = rbase + j
                s = urows[r, pl.ds(0, _L)] * irows[r, pl.ds(0, _L)]
                for cidx in range(1, NDC):
                    co = cidx * _L
                    s = s + urows[r, pl.ds(co, _L)] * irows[r, pl.ds(co, _L)]
                acc = acc + jnp.where(lane == j, jnp.sum(s), 0.0)
            out_v[pl.ds(rbase, _L)] = acc
            return carry

        lax.fori_loop(0, BPW // _L, group, 0)
        pltpu.sync_copy(out_v, out_hbm.at[pl.ds(base, BPW)])

    return mf


def kernel(user_ids, item_ids, user_table, item_table, user_bias, item_bias,
           global_bias):
    B = user_ids.shape[0]
    mf = _build(B, user_table.shape[1])
    return mf(user_ids.astype(jnp.int32), item_ids.astype(jnp.int32),
              user_table, item_table, user_bias.reshape(-1),
              item_bias.reshape(-1),
              jnp.broadcast_to(global_bias, (_L,)))
